# baseline (speedup 1.0000x reference)
import jax
import jax.numpy as jnp
from jax import lax
from jax.experimental import pallas as pl
from jax.experimental.pallas import tpu as pltpu
from jax.experimental.pallas import tpu_sc as plsc

N_NODES = 10000
D = 128
E = 320000

_info = plsc.get_sparse_core_info()
NC = _info.num_cores
NS = _info.num_subcores
L = _info.num_lanes
NW = NC * NS
EW = 10240
EP = NW * EW
CH = 40
NCH = EW // CH
NP = NCH // 2
NJ = D // L
VROWS = E // 8


NWRD = D // 2


def _sc_body(z_hbm, src_hbm, dst_hbm, out_hbm,
             sidx, didx, srowsA, drowsA, srowsB, drowsB,
             pbufA, pbufB, zsh, semA, semB, semOA, semOB):
    sid = lax.axis_index("s")
    wid = sid * NC + lax.axis_index("c")
    base = wid * EW
    for k in range(16):
        h = sid * 16 + k

        @pl.when(h < N_NODES // CH)
        def _():
            pltpu.sync_copy(z_hbm.at[pl.ds(h * CH, CH)], srowsA)
            pltpu.sync_copy(srowsA, zsh.at[pl.ds(h * CH, CH)])

    pltpu.sync_copy(src_hbm.at[pl.ds(base, EW)], sidx)
    pltpu.sync_copy(dst_hbm.at[pl.ds(base, EW)], didx)
    plsc.subcore_barrier()

    def fire(ci, sb, db, sem):
        sl = pl.ds(ci * CH, CH)

        @pl.when(ci % 3 == 0)
        def _():
            pltpu.async_copy(z_hbm.at[sidx.at[sl]], sb, sem)
            pltpu.async_copy(z_hbm.at[didx.at[sl]], db, sem)

        @pl.when(ci % 3 != 0)
        def _():
            pltpu.async_copy(zsh.at[sidx.at[sl]], sb, sem)
            pltpu.async_copy(zsh.at[didx.at[sl]], db, sem)

    def drain_gather(sb, db, sem):
        pltpu.make_async_copy(z_hbm.at[pl.ds(0, CH)], sb, sem).wait()
        pltpu.make_async_copy(z_hbm.at[pl.ds(0, CH)], db, sem).wait()

    def drain_out(pb, sem):
        pltpu.make_async_copy(pb, out_hbm.at[pl.ds(0, CH * L)], sem).wait()

    def compute(srows, drows, pbuf):
        for row in range(CH):
            acc = None
            for j in range(NJ):
                sv = srows[row, pl.ds(j * L, L)]
                dv = drows[row, pl.ds(j * L, L)]
                v = sv - dv + jnp.float32(1e-6)
                acc = v * v if acc is None else acc + v * v
            pbuf[pl.ds(row * L, L)] = acc

    fire(0, srowsA, drowsA, semA)

    def pair_body(h, carry):
        ci0 = h * 2
        ci1 = ci0 + 1
        fire(ci1, srowsB, drowsB, semB)
        drain_gather(srowsA, drowsA, semA)

        @pl.when(h > 0)
        def _():
            drain_out(pbufA, semOA)

        compute(srowsA, drowsA, pbufA)
        pltpu.async_copy(pbufA, out_hbm.at[pl.ds((base + ci0 * CH) * L, CH * L)],
                         semOA)

        @pl.when(h + 1 < NP)
        def _():
            fire(ci0 + 2, srowsA, drowsA, semA)

        drain_gather(srowsB, drowsB, semB)

        @pl.when(h > 0)
        def _():
            drain_out(pbufB, semOB)

        compute(srowsB, drowsB, pbufB)
        pltpu.async_copy(pbufB, out_hbm.at[pl.ds((base + ci1 * CH) * L, CH * L)],
                         semOB)
        return carry

    lax.fori_loop(0, NP, pair_body, 0)
    drain_out(pbufA, semOA)
    drain_out(pbufB, semOB)


@jax.jit
def _sc_partials(z, src_p, dst_p):
    mesh = plsc.VectorSubcoreMesh(core_axis_name="c", subcore_axis_name="s")
    return pl.kernel(
        _sc_body,
        mesh=mesh,
        out_type=jax.ShapeDtypeStruct((EP * L,), jnp.float32),
        scratch_types=[
            pltpu.VMEM((EW,), jnp.int32),
            pltpu.VMEM((EW,), jnp.int32),
            pltpu.VMEM((CH, D), jnp.float32),
            pltpu.VMEM((CH, D), jnp.float32),
            pltpu.VMEM((CH, D), jnp.float32),
            pltpu.VMEM((CH, D), jnp.float32),
            pltpu.VMEM((CH * L,), jnp.float32),
            pltpu.VMEM((CH * L,), jnp.float32),
            pltpu.VMEM_SHARED((N_NODES, D), jnp.float32),
            pltpu.SemaphoreType.DMA,
            pltpu.SemaphoreType.DMA,
            pltpu.SemaphoreType.DMA,
            pltpu.SemaphoreType.DMA,
        ],
    )(z, src_p, dst_p)


_FOLD_BLK = 256


def _fold_block(p3):
    b = p3.shape[0]
    jj = lax.broadcasted_iota(jnp.int32, (D, D), 0)
    cc = lax.broadcasted_iota(jnp.int32, (D, D), 1)
    acc = jnp.zeros((b, D), jnp.float32)
    for r in range(L):
        w_r = (cc == r * 8 + jj // L).astype(jnp.bfloat16)
        acc = acc + jnp.dot(p3[:, r, :].astype(jnp.bfloat16), w_r,
                            preferred_element_type=jnp.float32)
    return acc


def _tc_fold_body(p_ref, ss_ref, mn_ref):
    ss = _fold_block(p_ref[...])
    ss_ref[...] = ss
    i = pl.program_id(0)
    erow = i * _FOLD_BLK + lax.broadcasted_iota(jnp.int32, ss.shape, 0)
    valid = erow < E // D
    mn = jnp.min(jnp.where(valid, ss, jnp.float32(jnp.inf)))
    mn_ref[...] = jnp.full((1, 1, D), mn, jnp.float32)


def _tc_finish_body(ss_ref, mn_ref, o_ref):
    ss = ss_ref[...]
    m = 1.0 / jnp.sqrt(jnp.min(mn_ref[...]))
    rows = lax.broadcasted_iota(jnp.int32, ss.shape, 0)
    valid = rows < E // D
    dist = 1.0 / jnp.sqrt(ss)
    o_ref[...] = jnp.exp(jnp.where(valid, dist - m, 0.0))


def kernel(z, edge_index, p):
    src = edge_index[0].astype(jnp.int32)
    dst = edge_index[1].astype(jnp.int32)
    pad = EP - E
    src_p = jnp.concatenate([src, jnp.zeros((pad,), jnp.int32)])
    dst_p = jnp.concatenate([dst, jnp.ones((pad,), jnp.int32)])
    partials = _sc_partials(z, src_p, dst_p)
    nblk = EP // D // _FOLD_BLK
    ss, mns = pl.pallas_call(
        _tc_fold_body,
        grid=(nblk,),
        in_specs=[pl.BlockSpec((_FOLD_BLK, L, D), lambda i: (i, 0, 0))],
        out_specs=[pl.BlockSpec((_FOLD_BLK, D), lambda i: (i, 0)),
                   pl.BlockSpec((1, 1, D), lambda i: (i, 0, 0))],
        out_shape=[jax.ShapeDtypeStruct((EP // D, D), jnp.float32),
                   jax.ShapeDtypeStruct((nblk, 1, D), jnp.float32)],
    )(partials.reshape(EP // D, L, D))
    out = pl.pallas_call(
        _tc_finish_body,
        out_shape=jax.ShapeDtypeStruct((EP // D, D), jnp.float32),
    )(ss, mns)
    return out.reshape(EP)[:E]


# speedup vs baseline: 1.3407x; 1.3407x over previous
import jax
import jax.numpy as jnp
from jax import lax
from jax.experimental import pallas as pl
from jax.experimental.pallas import tpu as pltpu
from jax.experimental.pallas import tpu_sc as plsc

N_NODES = 10000
D = 128
E = 320000

_info = plsc.get_sparse_core_info()
NC = _info.num_cores
NS = _info.num_subcores
L = _info.num_lanes
NW = NC * NS
EW = 10240
EP = NW * EW
CH = 40
NCH = EW // CH
NP = NCH // 2
NJ = D // L
VROWS = E // 8


NWRD = D // 2


def _sc_body(z_hbm, src_hbm, dst_hbm, out_hbm,
             sidx, didx, srowsA, drowsA, srowsB, drowsB,
             pbufA, pbufB, zsh, semA, semB, semOA, semOB):
    sid = lax.axis_index("s")
    wid = sid * NC + lax.axis_index("c")
    base = wid * EW
    for k in range(16):
        h = sid * 16 + k

        @pl.when(h < N_NODES // CH)
        def _():
            pltpu.sync_copy(z_hbm.at[pl.ds(h * CH, CH)], srowsA)
            pltpu.sync_copy(srowsA, zsh.at[pl.ds(h * CH, CH)])

    pltpu.sync_copy(src_hbm.at[pl.ds(base, EW)], sidx)
    pltpu.sync_copy(dst_hbm.at[pl.ds(base, EW)], didx)
    plsc.subcore_barrier()

    def fire(ci, sb, db, sem):
        sl = pl.ds(ci * CH, CH)
        pltpu.async_copy(zsh.at[sidx.at[sl]], sb, sem)
        pltpu.async_copy(zsh.at[didx.at[sl]], db, sem)

    def drain_gather(sb, db, sem):
        pltpu.make_async_copy(z_hbm.at[pl.ds(0, CH)], sb, sem).wait()
        pltpu.make_async_copy(z_hbm.at[pl.ds(0, CH)], db, sem).wait()

    def drain_out(pb, sem):
        pltpu.make_async_copy(pb, out_hbm.at[pl.ds(0, CH * L)], sem).wait()

    def compute(srows, drows, pbuf):
        for row in range(CH):
            acc = None
            for j in range(NJ):
                sv = srows[row, pl.ds(j * L, L)]
                dv = drows[row, pl.ds(j * L, L)]
                v = sv - dv + jnp.float32(1e-6)
                acc = v * v if acc is None else acc + v * v
            pbuf[pl.ds(row * L, L)] = acc

    fire(0, srowsA, drowsA, semA)

    def pair_body(h, carry):
        ci0 = h * 2
        ci1 = ci0 + 1
        fire(ci1, srowsB, drowsB, semB)
        drain_gather(srowsA, drowsA, semA)

        @pl.when(h > 0)
        def _():
            drain_out(pbufA, semOA)

        compute(srowsA, drowsA, pbufA)
        pltpu.async_copy(pbufA, out_hbm.at[pl.ds((base + ci0 * CH) * L, CH * L)],
                         semOA)

        @pl.when(h + 1 < NP)
        def _():
            fire(ci0 + 2, srowsA, drowsA, semA)

        drain_gather(srowsB, drowsB, semB)

        @pl.when(h > 0)
        def _():
            drain_out(pbufB, semOB)

        compute(srowsB, drowsB, pbufB)
        pltpu.async_copy(pbufB, out_hbm.at[pl.ds((base + ci1 * CH) * L, CH * L)],
                         semOB)
        return carry

    lax.fori_loop(0, NP, pair_body, 0)
    drain_out(pbufA, semOA)
    drain_out(pbufB, semOB)


@jax.jit
def _sc_partials(z, src_p, dst_p):
    mesh = plsc.VectorSubcoreMesh(core_axis_name="c", subcore_axis_name="s")
    return pl.kernel(
        _sc_body,
        mesh=mesh,
        out_type=jax.ShapeDtypeStruct((EP * L,), jnp.float32),
        scratch_types=[
            pltpu.VMEM((EW,), jnp.int32),
            pltpu.VMEM((EW,), jnp.int32),
            pltpu.VMEM((CH, D), jnp.float32),
            pltpu.VMEM((CH, D), jnp.float32),
            pltpu.VMEM((CH, D), jnp.float32),
            pltpu.VMEM((CH, D), jnp.float32),
            pltpu.VMEM((CH * L,), jnp.float32),
            pltpu.VMEM((CH * L,), jnp.float32),
            pltpu.VMEM_SHARED((N_NODES, D), jnp.float32),
            pltpu.SemaphoreType.DMA,
            pltpu.SemaphoreType.DMA,
            pltpu.SemaphoreType.DMA,
            pltpu.SemaphoreType.DMA,
        ],
    )(z, src_p, dst_p)


_FOLD_BLK = 256


def _fold_block(p3):
    b = p3.shape[0]
    jj = lax.broadcasted_iota(jnp.int32, (D, D), 0)
    cc = lax.broadcasted_iota(jnp.int32, (D, D), 1)
    acc = jnp.zeros((b, D), jnp.float32)
    for r in range(L):
        w_r = (cc == r * 8 + jj // L).astype(jnp.bfloat16)
        acc = acc + jnp.dot(p3[:, r, :].astype(jnp.bfloat16), w_r,
                            preferred_element_type=jnp.float32)
    return acc


def _tc_fold_body(p_ref, ss_ref, mn_ref):
    ss = _fold_block(p_ref[...])
    ss_ref[...] = ss
    i = pl.program_id(0)
    erow = i * _FOLD_BLK + lax.broadcasted_iota(jnp.int32, ss.shape, 0)
    valid = erow < E // D
    mn = jnp.min(jnp.where(valid, ss, jnp.float32(jnp.inf)))
    mn_ref[...] = jnp.full((1, 1, D), mn, jnp.float32)


def _tc_finish_body(ss_ref, mn_ref, o_ref):
    ss = ss_ref[...]
    m = 1.0 / jnp.sqrt(jnp.min(mn_ref[...]))
    rows = lax.broadcasted_iota(jnp.int32, ss.shape, 0)
    valid = rows < E // D
    dist = 1.0 / jnp.sqrt(ss)
    o_ref[...] = jnp.exp(jnp.where(valid, dist - m, 0.0))


def kernel(z, edge_index, p):
    src = edge_index[0].astype(jnp.int32)
    dst = edge_index[1].astype(jnp.int32)
    pad = EP - E
    src_p = jnp.concatenate([src, jnp.zeros((pad,), jnp.int32)])
    dst_p = jnp.concatenate([dst, jnp.ones((pad,), jnp.int32)])
    partials = _sc_partials(z, src_p, dst_p)
    nblk = EP // D // _FOLD_BLK
    ss, mns = pl.pallas_call(
        _tc_fold_body,
        grid=(nblk,),
        in_specs=[pl.BlockSpec((_FOLD_BLK, L, D), lambda i: (i, 0, 0))],
        out_specs=[pl.BlockSpec((_FOLD_BLK, D), lambda i: (i, 0)),
                   pl.BlockSpec((1, 1, D), lambda i: (i, 0, 0))],
        out_shape=[jax.ShapeDtypeStruct((EP // D, D), jnp.float32),
                   jax.ShapeDtypeStruct((nblk, 1, D), jnp.float32)],
    )(partials.reshape(EP // D, L, D))
    out = pl.pallas_call(
        _tc_finish_body,
        out_shape=jax.ShapeDtypeStruct((EP // D, D), jnp.float32),
    )(ss, mns)
    return out.reshape(EP)[:E]


# speedup vs baseline: 2.1041x; 1.5694x over previous
import jax
import jax.numpy as jnp
from jax import lax
from jax.experimental import pallas as pl
from jax.experimental.pallas import tpu as pltpu
from jax.experimental.pallas import tpu_sc as plsc

N_NODES = 10000
D = 128
E = 320000

_info = plsc.get_sparse_core_info()
NC = _info.num_cores
NS = _info.num_subcores
L = _info.num_lanes
NW = NC * NS
EW = 10240
EP = NW * EW
CH = 32
NCH = EW // CH
NP = NCH // 2
NJ = D // L
VROWS = E // 8


NWRD = D // 2


def _sc_body(z_hbm, src_hbm, dst_hbm, out_hbm,
             sidx, didx, srowsA, drowsA, srowsB, drowsB,
             pbufA, pbufB, zstage, zsh, semA, semB, semOA, semOB):
    sid = lax.axis_index("s")
    wid = sid * NC + lax.axis_index("c")
    base = wid * EW
    for k in range(16):
        h = sid * 16 + k

        @pl.when(h < N_NODES // 40)
        def _():
            pltpu.sync_copy(z_hbm.at[pl.ds(h * 40, 40)], zstage)
            pltpu.sync_copy(zstage, zsh.at[pl.ds(h * 40, 40)])

    pltpu.sync_copy(src_hbm.at[pl.ds(base, EW)], sidx)
    pltpu.sync_copy(dst_hbm.at[pl.ds(base, EW)], didx)
    plsc.subcore_barrier()

    def fire(ci, sb, db, sem):
        h = CH // 2
        sl0 = pl.ds(ci * CH, h)
        sl1 = pl.ds(ci * CH + h, h)
        pltpu.async_copy(zsh.at[sidx.at[sl0]], sb.at[pl.ds(0, h)], sem)
        pltpu.async_copy(zsh.at[sidx.at[sl1]], sb.at[pl.ds(h, h)], sem)
        pltpu.async_copy(zsh.at[didx.at[sl0]], db.at[pl.ds(0, h)], sem)
        pltpu.async_copy(zsh.at[didx.at[sl1]], db.at[pl.ds(h, h)], sem)

    def drain_gather(sb, db, sem):
        pltpu.make_async_copy(z_hbm.at[pl.ds(0, CH)], sb, sem).wait()
        pltpu.make_async_copy(z_hbm.at[pl.ds(0, CH)], db, sem).wait()

    def drain_out(pb, sem):
        pltpu.make_async_copy(pb, out_hbm.at[pl.ds(0, CH * L)], sem).wait()

    def compute(srows, drows, pbuf):
        for row in range(CH):
            acc = None
            for j in range(NJ):
                sv = srows[row, pl.ds(j * L, L)]
                dv = drows[row, pl.ds(j * L, L)]
                v = sv - dv + jnp.float32(1e-6)
                acc = v * v if acc is None else acc + v * v
            pbuf[pl.ds(row * L, L)] = acc

    fire(0, srowsA, drowsA, semA)

    def pair_body(h, carry):
        ci0 = h * 2
        ci1 = ci0 + 1
        fire(ci1, srowsB, drowsB, semB)
        drain_gather(srowsA, drowsA, semA)

        @pl.when(h > 0)
        def _():
            drain_out(pbufA, semOA)

        compute(srowsA, drowsA, pbufA)
        pltpu.async_copy(pbufA, out_hbm.at[pl.ds((base + ci0 * CH) * L, CH * L)],
                         semOA)

        @pl.when(h + 1 < NP)
        def _():
            fire(ci0 + 2, srowsA, drowsA, semA)

        drain_gather(srowsB, drowsB, semB)

        @pl.when(h > 0)
        def _():
            drain_out(pbufB, semOB)

        compute(srowsB, drowsB, pbufB)
        pltpu.async_copy(pbufB, out_hbm.at[pl.ds((base + ci1 * CH) * L, CH * L)],
                         semOB)
        return carry

    lax.fori_loop(0, NP, pair_body, 0)
    drain_out(pbufA, semOA)
    drain_out(pbufB, semOB)


@jax.jit
def _sc_partials(z, src_p, dst_p):
    mesh = plsc.VectorSubcoreMesh(core_axis_name="c", subcore_axis_name="s")
    return pl.kernel(
        _sc_body,
        mesh=mesh,
        out_type=jax.ShapeDtypeStruct((EP * L,), jnp.float32),
        scratch_types=[
            pltpu.VMEM((EW,), jnp.int32),
            pltpu.VMEM((EW,), jnp.int32),
            pltpu.VMEM((CH, D), jnp.float32),
            pltpu.VMEM((CH, D), jnp.float32),
            pltpu.VMEM((CH, D), jnp.float32),
            pltpu.VMEM((CH, D), jnp.float32),
            pltpu.VMEM((CH * L,), jnp.float32),
            pltpu.VMEM((CH * L,), jnp.float32),
            pltpu.VMEM((40, D), jnp.float32),
            pltpu.VMEM_SHARED((N_NODES, D), jnp.float32),
            pltpu.SemaphoreType.DMA,
            pltpu.SemaphoreType.DMA,
            pltpu.SemaphoreType.DMA,
            pltpu.SemaphoreType.DMA,
        ],
    )(z, src_p, dst_p)


_FOLD_BLK = 256


def _fold_block(p3):
    b = p3.shape[0]
    jj = lax.broadcasted_iota(jnp.int32, (D, D), 0)
    cc = lax.broadcasted_iota(jnp.int32, (D, D), 1)
    acc = jnp.zeros((b, D), jnp.float32)
    for r in range(L):
        w_r = (cc == r * 8 + jj // L).astype(jnp.bfloat16)
        acc = acc + jnp.dot(p3[:, r, :].astype(jnp.bfloat16), w_r,
                            preferred_element_type=jnp.float32)
    return acc


def _tc_fold_body(p_ref, ss_ref, mn_ref):
    ss = _fold_block(p_ref[...])
    ss_ref[...] = ss
    i = pl.program_id(0)
    erow = i * _FOLD_BLK + lax.broadcasted_iota(jnp.int32, ss.shape, 0)
    valid = erow < E // D
    mn = jnp.min(jnp.where(valid, ss, jnp.float32(jnp.inf)))
    mn_ref[...] = jnp.full((1, 1, D), mn, jnp.float32)


def _tc_finish_body(ss_ref, mn_ref, o_ref):
    ss = ss_ref[...]
    m = 1.0 / jnp.sqrt(jnp.min(mn_ref[...]))
    rows = lax.broadcasted_iota(jnp.int32, ss.shape, 0)
    valid = rows < E // D
    dist = 1.0 / jnp.sqrt(ss)
    o_ref[...] = jnp.exp(jnp.where(valid, dist - m, 0.0))


def kernel(z, edge_index, p):
    src = edge_index[0].astype(jnp.int32)
    dst = edge_index[1].astype(jnp.int32)
    pad = EP - E
    src_p = jnp.concatenate([src, jnp.zeros((pad,), jnp.int32)])
    dst_p = jnp.concatenate([dst, jnp.ones((pad,), jnp.int32)])
    partials = _sc_partials(z, src_p, dst_p)
    nblk = EP // D // _FOLD_BLK
    ss, mns = pl.pallas_call(
        _tc_fold_body,
        grid=(nblk,),
        in_specs=[pl.BlockSpec((_FOLD_BLK, L, D), lambda i: (i, 0, 0))],
        out_specs=[pl.BlockSpec((_FOLD_BLK, D), lambda i: (i, 0)),
                   pl.BlockSpec((1, 1, D), lambda i: (i, 0, 0))],
        out_shape=[jax.ShapeDtypeStruct((EP // D, D), jnp.float32),
                   jax.ShapeDtypeStruct((nblk, 1, D), jnp.float32)],
    )(partials.reshape(EP // D, L, D))
    out = pl.pallas_call(
        _tc_finish_body,
        out_shape=jax.ShapeDtypeStruct((EP // D, D), jnp.float32),
    )(ss, mns)
    return out.reshape(EP)[:E]


# speedup vs baseline: 2.1201x; 1.0076x over previous
import jax
import jax.numpy as jnp
from jax import lax
from jax.experimental import pallas as pl
from jax.experimental.pallas import tpu as pltpu
from jax.experimental.pallas import tpu_sc as plsc

N_NODES = 10000
D = 128
E = 320000

_info = plsc.get_sparse_core_info()
NC = _info.num_cores
NS = _info.num_subcores
L = _info.num_lanes
NW = NC * NS
EW = 10240
EP = NW * EW
CH = 32
NCH = EW // CH
NP = NCH // 2
NJ = D // L
VROWS = E // 8


NWRD = D // 2


def _sc_body(z_hbm, src_hbm, dst_hbm, out_hbm,
             sidx, didx, srowsA, drowsA, srowsB, drowsB,
             pbufA, pbufB, zstage, zsh, semA, semB, semOA, semOB):
    sid = lax.axis_index("s")
    wid = sid * NC + lax.axis_index("c")
    base = wid * EW
    for k in range(16):
        h = sid * 16 + k

        @pl.when(h < N_NODES // 40)
        def _():
            pltpu.sync_copy(z_hbm.at[pl.ds(h * 40, 40)], zstage)
            pltpu.sync_copy(zstage, zsh.at[pl.ds(h * 40, 40)])

    pltpu.sync_copy(src_hbm.at[pl.ds(base, EW)], sidx)
    pltpu.sync_copy(dst_hbm.at[pl.ds(base, EW)], didx)
    plsc.subcore_barrier()

    NSPL = 4

    def fire(ci, sb, db, sem):
        h = CH // NSPL
        for k in range(NSPL):
            slk = pl.ds(ci * CH + k * h, h)
            dstk = pl.ds(k * h, h)
            pltpu.async_copy(zsh.at[sidx.at[slk]], sb.at[dstk], sem)
            pltpu.async_copy(zsh.at[didx.at[slk]], db.at[dstk], sem)

    def drain_gather(sb, db, sem):
        pltpu.make_async_copy(z_hbm.at[pl.ds(0, CH)], sb, sem).wait()
        pltpu.make_async_copy(z_hbm.at[pl.ds(0, CH)], db, sem).wait()

    def drain_out(pb, sem):
        pltpu.make_async_copy(pb, out_hbm.at[pl.ds(0, CH * L)], sem).wait()

    def compute(srows, drows, pbuf):
        for row in range(CH):
            acc = None
            for j in range(NJ):
                sv = srows[row, pl.ds(j * L, L)]
                dv = drows[row, pl.ds(j * L, L)]
                v = sv - dv + jnp.float32(1e-6)
                acc = v * v if acc is None else acc + v * v
            pbuf[pl.ds(row * L, L)] = acc

    fire(0, srowsA, drowsA, semA)

    def pair_body(h, carry):
        ci0 = h * 2
        ci1 = ci0 + 1
        fire(ci1, srowsB, drowsB, semB)
        drain_gather(srowsA, drowsA, semA)

        @pl.when(h > 0)
        def _():
            drain_out(pbufA, semOA)

        compute(srowsA, drowsA, pbufA)
        pltpu.async_copy(pbufA, out_hbm.at[pl.ds((base + ci0 * CH) * L, CH * L)],
                         semOA)

        @pl.when(h + 1 < NP)
        def _():
            fire(ci0 + 2, srowsA, drowsA, semA)

        drain_gather(srowsB, drowsB, semB)

        @pl.when(h > 0)
        def _():
            drain_out(pbufB, semOB)

        compute(srowsB, drowsB, pbufB)
        pltpu.async_copy(pbufB, out_hbm.at[pl.ds((base + ci1 * CH) * L, CH * L)],
                         semOB)
        return carry

    lax.fori_loop(0, NP, pair_body, 0)
    drain_out(pbufA, semOA)
    drain_out(pbufB, semOB)


@jax.jit
def _sc_partials(z, src_p, dst_p):
    mesh = plsc.VectorSubcoreMesh(core_axis_name="c", subcore_axis_name="s")
    return pl.kernel(
        _sc_body,
        mesh=mesh,
        out_type=jax.ShapeDtypeStruct((EP * L,), jnp.float32),
        scratch_types=[
            pltpu.VMEM((EW,), jnp.int32),
            pltpu.VMEM((EW,), jnp.int32),
            pltpu.VMEM((CH, D), jnp.float32),
            pltpu.VMEM((CH, D), jnp.float32),
            pltpu.VMEM((CH, D), jnp.float32),
            pltpu.VMEM((CH, D), jnp.float32),
            pltpu.VMEM((CH * L,), jnp.float32),
            pltpu.VMEM((CH * L,), jnp.float32),
            pltpu.VMEM((40, D), jnp.float32),
            pltpu.VMEM_SHARED((N_NODES, D), jnp.float32),
            pltpu.SemaphoreType.DMA,
            pltpu.SemaphoreType.DMA,
            pltpu.SemaphoreType.DMA,
            pltpu.SemaphoreType.DMA,
        ],
    )(z, src_p, dst_p)


_FOLD_BLK = 256


def _fold_block(p3):
    b = p3.shape[0]
    jj = lax.broadcasted_iota(jnp.int32, (D, D), 0)
    cc = lax.broadcasted_iota(jnp.int32, (D, D), 1)
    acc = jnp.zeros((b, D), jnp.float32)
    for r in range(L):
        w_r = (cc == r * 8 + jj // L).astype(jnp.bfloat16)
        acc = acc + jnp.dot(p3[:, r, :].astype(jnp.bfloat16), w_r,
                            preferred_element_type=jnp.float32)
    return acc


def _tc_fold_body(p_ref, ss_ref, mn_ref):
    ss = _fold_block(p_ref[...])
    ss_ref[...] = ss
    i = pl.program_id(0)
    erow = i * _FOLD_BLK + lax.broadcasted_iota(jnp.int32, ss.shape, 0)
    valid = erow < E // D
    mn = jnp.min(jnp.where(valid, ss, jnp.float32(jnp.inf)))
    mn_ref[...] = jnp.full((1, 1, D), mn, jnp.float32)


def _tc_finish_body(ss_ref, mn_ref, o_ref):
    ss = ss_ref[...]
    m = 1.0 / jnp.sqrt(jnp.min(mn_ref[...]))
    rows = lax.broadcasted_iota(jnp.int32, ss.shape, 0)
    valid = rows < E // D
    dist = 1.0 / jnp.sqrt(ss)
    o_ref[...] = jnp.exp(jnp.where(valid, dist - m, 0.0))


def kernel(z, edge_index, p):
    src = edge_index[0].astype(jnp.int32)
    dst = edge_index[1].astype(jnp.int32)
    pad = EP - E
    src_p = jnp.concatenate([src, jnp.zeros((pad,), jnp.int32)])
    dst_p = jnp.concatenate([dst, jnp.ones((pad,), jnp.int32)])
    partials = _sc_partials(z, src_p, dst_p)
    nblk = EP // D // _FOLD_BLK
    ss, mns = pl.pallas_call(
        _tc_fold_body,
        grid=(nblk,),
        in_specs=[pl.BlockSpec((_FOLD_BLK, L, D), lambda i: (i, 0, 0))],
        out_specs=[pl.BlockSpec((_FOLD_BLK, D), lambda i: (i, 0)),
                   pl.BlockSpec((1, 1, D), lambda i: (i, 0, 0))],
        out_shape=[jax.ShapeDtypeStruct((EP // D, D), jnp.float32),
                   jax.ShapeDtypeStruct((nblk, 1, D), jnp.float32)],
    )(partials.reshape(EP // D, L, D))
    out = pl.pallas_call(
        _tc_finish_body,
        out_shape=jax.ShapeDtypeStruct((EP // D, D), jnp.float32),
    )(ss, mns)
    return out.reshape(EP)[:E]


# speedup vs baseline: 2.1207x; 1.0003x over previous
import jax
import jax.numpy as jnp
from jax import lax
from jax.experimental import pallas as pl
from jax.experimental.pallas import tpu as pltpu
from jax.experimental.pallas import tpu_sc as plsc

N_NODES = 10000
D = 128
E = 320000

_info = plsc.get_sparse_core_info()
NC = _info.num_cores
NS = _info.num_subcores
L = _info.num_lanes
NW = NC * NS
EW = 10240
EP = NW * EW
CH = 32
NCH = EW // CH
NP = NCH // 2
NJ = D // L


def _sc_body(z_hbm, src_hbm, dst_hbm, out_hbm,
             sidx, didx, srowsA, drowsA, srowsB, drowsB,
             pbufA, pbufB, zstage, zsh, semA, semB, semOA, semOB):
    sid = lax.axis_index("s")
    wid = sid * NC + lax.axis_index("c")
    base = wid * EW
    for k in range(16):
        h = sid * 16 + k

        @pl.when(h < N_NODES // 40)
        def _():
            pltpu.sync_copy(z_hbm.at[pl.ds(h * 40, 40)], zstage)
            pltpu.sync_copy(zstage, zsh.at[pl.ds(h * 40, 40)])

    pltpu.sync_copy(src_hbm.at[pl.ds(base, EW)], sidx)
    pltpu.sync_copy(dst_hbm.at[pl.ds(base, EW)], didx)
    plsc.subcore_barrier()

    NSPL = 4

    def fire(ci, sb, db, sem):
        h = CH // NSPL
        for k in range(NSPL):
            slk = pl.ds(ci * CH + k * h, h)
            dstk = pl.ds(k * h, h)
            pltpu.async_copy(zsh.at[sidx.at[slk]], sb.at[dstk], sem)
            pltpu.async_copy(zsh.at[didx.at[slk]], db.at[dstk], sem)

    def drain_gather(sb, db, sem):
        pltpu.make_async_copy(z_hbm.at[pl.ds(0, CH)], sb, sem).wait()
        pltpu.make_async_copy(z_hbm.at[pl.ds(0, CH)], db, sem).wait()

    def drain_out(pb, sem):
        pltpu.make_async_copy(pb, out_hbm.at[pl.ds(0, CH * L)], sem).wait()

    def compute(srows, drows, pbuf):
        for row in range(CH):
            acc = None
            for j in range(NJ):
                sv = srows[row, pl.ds(j * L, L)]
                dv = drows[row, pl.ds(j * L, L)]
                v = sv - dv + jnp.float32(1e-6)
                acc = v * v if acc is None else acc + v * v
            pbuf[pl.ds(row * L, L)] = acc

    fire(0, srowsA, drowsA, semA)

    def pair_body(h, carry):
        ci0 = h * 2
        ci1 = ci0 + 1
        fire(ci1, srowsB, drowsB, semB)
        drain_gather(srowsA, drowsA, semA)

        @pl.when(h > 0)
        def _():
            drain_out(pbufA, semOA)

        compute(srowsA, drowsA, pbufA)
        pltpu.async_copy(pbufA, out_hbm.at[pl.ds((base + ci0 * CH) * L, CH * L)],
                         semOA)

        @pl.when(h + 1 < NP)
        def _():
            fire(ci0 + 2, srowsA, drowsA, semA)

        drain_gather(srowsB, drowsB, semB)

        @pl.when(h > 0)
        def _():
            drain_out(pbufB, semOB)

        compute(srowsB, drowsB, pbufB)
        pltpu.async_copy(pbufB, out_hbm.at[pl.ds((base + ci1 * CH) * L, CH * L)],
                         semOB)
        return carry

    lax.fori_loop(0, NP, pair_body, 0)
    drain_out(pbufA, semOA)
    drain_out(pbufB, semOB)


@jax.jit
def _sc_partials(z, src_p, dst_p):
    mesh = plsc.VectorSubcoreMesh(core_axis_name="c", subcore_axis_name="s")
    return pl.kernel(
        _sc_body,
        mesh=mesh,
        out_type=jax.ShapeDtypeStruct((EP * L,), jnp.float32),
        scratch_types=[
            pltpu.VMEM((EW,), jnp.int32),
            pltpu.VMEM((EW,), jnp.int32),
            pltpu.VMEM((CH, D), jnp.float32),
            pltpu.VMEM((CH, D), jnp.float32),
            pltpu.VMEM((CH, D), jnp.float32),
            pltpu.VMEM((CH, D), jnp.float32),
            pltpu.VMEM((CH * L,), jnp.float32),
            pltpu.VMEM((CH * L,), jnp.float32),
            pltpu.VMEM((40, D), jnp.float32),
            pltpu.VMEM_SHARED((N_NODES, D), jnp.float32),
            pltpu.SemaphoreType.DMA,
            pltpu.SemaphoreType.DMA,
            pltpu.SemaphoreType.DMA,
            pltpu.SemaphoreType.DMA,
        ],
    )(z, src_p, dst_p)


_FOLD_BLK = 256


def _fold_block(p3):
    b = p3.shape[0]
    jj = lax.broadcasted_iota(jnp.int32, (D, D), 0)
    cc = lax.broadcasted_iota(jnp.int32, (D, D), 1)
    acc = jnp.zeros((b, D), jnp.float32)
    for r in range(L):
        w_r = (cc == r * 8 + jj // L).astype(jnp.bfloat16)
        acc = acc + jnp.dot(p3[:, r, :].astype(jnp.bfloat16), w_r,
                            preferred_element_type=jnp.float32)
    return acc


def _tc_fold_body(p_ref, ss_ref, mn_ref):
    ss = _fold_block(p_ref[...])
    ss_ref[...] = ss
    i = pl.program_id(0)
    erow = i * _FOLD_BLK + lax.broadcasted_iota(jnp.int32, ss.shape, 0)
    valid = erow < E // D
    mn = jnp.min(jnp.where(valid, ss, jnp.float32(jnp.inf)))
    mn_ref[...] = jnp.full((1, 1, D), mn, jnp.float32)


def _tc_finish_body(ss_ref, mn_ref, o_ref):
    ss = ss_ref[...]
    m = 1.0 / jnp.sqrt(jnp.min(mn_ref[...]))
    rows = lax.broadcasted_iota(jnp.int32, ss.shape, 0)
    valid = rows < E // D
    dist = 1.0 / jnp.sqrt(ss)
    o_ref[...] = jnp.exp(jnp.where(valid, dist - m, 0.0))


def kernel(z, edge_index, p):
    src = edge_index[0].astype(jnp.int32)
    dst = edge_index[1].astype(jnp.int32)
    pad = EP - E
    src_p = jnp.concatenate([src, jnp.zeros((pad,), jnp.int32)])
    dst_p = jnp.concatenate([dst, jnp.ones((pad,), jnp.int32)])
    partials = _sc_partials(z, src_p, dst_p)
    nblk = EP // D // _FOLD_BLK
    ss, mns = pl.pallas_call(
        _tc_fold_body,
        grid=(nblk,),
        in_specs=[pl.BlockSpec((_FOLD_BLK, L, D), lambda i: (i, 0, 0))],
        out_specs=[pl.BlockSpec((_FOLD_BLK, D), lambda i: (i, 0)),
                   pl.BlockSpec((1, 1, D), lambda i: (i, 0, 0))],
        out_shape=[jax.ShapeDtypeStruct((EP // D, D), jnp.float32),
                   jax.ShapeDtypeStruct((nblk, 1, D), jnp.float32)],
    )(partials.reshape(EP // D, L, D))
    out = pl.pallas_call(
        _tc_finish_body,
        out_shape=jax.ShapeDtypeStruct((EP // D, D), jnp.float32),
    )(ss, mns)
    return out.reshape(EP)[:E]
